# baseline (speedup 1.0000x reference)
import functools

import jax
import jax.numpy as jnp
from jax import lax
from jax.experimental import pallas as pl
from jax.experimental.pallas import tpu as pltpu
from jax.experimental.pallas import tpu_sc as plsc

C = 8
N = 32768
TIME = 512
BATCH = 8192

NC = 2
NS = 16
NW = NC * NS
ROWS = C * BATCH
RPW = ROWS // NW
WPC = NW // C

SHC = 128
CHUNK = 512
NCHUNK = RPW // CHUNK
CWLEN = (CHUNK - SHC) + TIME
NQ = CHUNK // SHC
NL = CWLEN // SHC
RAWC = CHUNK + TIME

_mesh = plsc.VectorSubcoreMesh(core_axis_name="c", subcore_axis_name="s")


@functools.partial(
    pl.kernel,
    out_type=jax.ShapeDtypeStruct((C, BATCH, TIME), jnp.float32),
    mesh=_mesh,
    scratch_types=[
        pltpu.VMEM((RAWC,), jnp.float32),
        pltpu.VMEM((RAWC,), jnp.float32),
        pltpu.VMEM((SHC, CWLEN), jnp.float32),
        pltpu.SemaphoreType.DMA,
        pltpu.SemaphoreType.DMA,
        pltpu.SemaphoreType.DMA,
        pltpu.SemaphoreType.DMA,
        pltpu.SemaphoreType.DMA,
    ],
    compiler_params=pltpu.CompilerParams(use_tc_tiling_on_sc=True),
)
def _hankel_sc(x_hbm, out_hbm, raw_a, raw_b, win_v, in_sem, s0, s1, s2, s3):
    qsem = (s0, s1, s2, s3)
    raws = (raw_a, raw_b)
    wid = lax.axis_index("s") * NC + lax.axis_index("c")
    chan = wid // WPC
    b0 = (wid % WPC) * RPW
    base = chan * N + b0

    def drain(sem):
        pltpu.make_async_copy(
            win_v.at[pl.ds(0, SHC), pl.ds(0, TIME)],
            out_hbm.at[chan, pl.ds(b0, SHC)],
            sem,
        ).wait()

    pltpu.async_copy(x_hbm.at[pl.ds(base, RAWC)], raw_a, in_sem)

    for ck in range(NCHUNK):
        rawc = raws[ck % 2]
        pltpu.make_async_copy(
            x_hbm.at[pl.ds(base, RAWC)], rawc, in_sem
        ).wait()
        if ck + 1 < NCHUNK:
            pltpu.async_copy(
                x_hbm.at[pl.ds(base + CHUNK * (ck + 1), RAWC)],
                raws[(ck + 1) % 2],
                in_sem,
            )

        for L in range(NL):
            if ck > 0 and L < NQ:
                drain(qsem[L])

            @plsc.parallel_loop(0, SHC, unroll=8)
            def build(j):
                for m in range(SHC // 16):
                    win_v[j, pl.ds(128 * L + 16 * m, 16)] = rawc[
                        pl.ds(j + 128 * L + 16 * m, 16)
                    ]

            if L >= NL - NQ:
                qq = L - (NL - NQ)
                pltpu.async_copy(
                    win_v.at[pl.ds(0, SHC), pl.ds(128 * qq, TIME)],
                    out_hbm.at[chan, pl.ds(b0 + CHUNK * ck + SHC * qq, SHC)],
                    qsem[qq],
                )

    for qq in range(NQ):
        drain(qsem[qq])


def kernel(x):
    return _hankel_sc(x.reshape(-1))


# speedup vs baseline: 1.0178x; 1.0178x over previous
import functools

import jax
import jax.numpy as jnp
from jax import lax
from jax.experimental import pallas as pl
from jax.experimental.pallas import tpu as pltpu
from jax.experimental.pallas import tpu_sc as plsc

C = 8
N = 32768
TIME = 512
BATCH = 8192

NC = 2
NS = 16
NW = NC * NS
ROWS = C * BATCH
RPW = ROWS // NW
WPC = NW // C

SHC = 128
CHUNK = 512
NCHUNK = RPW // CHUNK
CWLEN = (CHUNK - SHC) + TIME
NQ = CHUNK // SHC
NL = CWLEN // SHC
RAWC = CHUNK + TIME

_mesh = plsc.VectorSubcoreMesh(core_axis_name="c", subcore_axis_name="s")


@functools.partial(
    pl.kernel,
    out_type=jax.ShapeDtypeStruct((C, BATCH, TIME), jnp.float32),
    mesh=_mesh,
    scratch_types=[
        pltpu.VMEM((RAWC,), jnp.float32),
        pltpu.VMEM((RAWC,), jnp.float32),
        pltpu.VMEM((SHC, CWLEN), jnp.float32),
        pltpu.SemaphoreType.DMA,
        pltpu.SemaphoreType.DMA,
        pltpu.SemaphoreType.DMA,
        pltpu.SemaphoreType.DMA,
        pltpu.SemaphoreType.DMA,
    ],
    compiler_params=pltpu.CompilerParams(use_tc_tiling_on_sc=True),
)
def _hankel_sc(x_hbm, out_hbm, raw_a, raw_b, win_v, in_sem, s0, s1, s2, s3):
    qsem = (s0, s1, s2, s3)
    raws = (raw_a, raw_b)
    wid = lax.axis_index("s") * NC + lax.axis_index("c")
    chan = wid // WPC
    b0 = (wid % WPC) * RPW
    base = chan * N + b0

    def drain(sem):
        pltpu.make_async_copy(
            win_v.at[pl.ds(0, SHC), pl.ds(0, TIME)],
            out_hbm.at[chan, pl.ds(b0, SHC)],
            sem,
        ).wait()

    pltpu.async_copy(x_hbm.at[pl.ds(base, RAWC)], raw_a, in_sem)

    for ck in range(NCHUNK):
        rawc = raws[ck % 2]
        pltpu.make_async_copy(
            x_hbm.at[pl.ds(base, RAWC)], rawc, in_sem
        ).wait()
        if ck + 1 < NCHUNK:
            pltpu.async_copy(
                x_hbm.at[pl.ds(base + CHUNK * (ck + 1), RAWC)],
                raws[(ck + 1) % 2],
                in_sem,
            )

        for L in range(NL):
            if ck > 0 and L < NQ:
                drain(qsem[L])

            @plsc.parallel_loop(0, SHC, unroll=4)
            def build(j):
                for m in range(SHC // 16):
                    win_v[j, pl.ds(128 * L + 16 * m, 16)] = rawc[
                        pl.ds(j + 128 * L + 16 * m, 16)
                    ]

            if L >= NL - NQ:
                qq = L - (NL - NQ)
                pltpu.async_copy(
                    win_v.at[pl.ds(0, SHC), pl.ds(128 * qq, TIME)],
                    out_hbm.at[chan, pl.ds(b0 + CHUNK * ck + SHC * qq, SHC)],
                    qsem[qq],
                )

    for qq in range(NQ):
        drain(qsem[qq])


def kernel(x):
    return _hankel_sc(x.reshape(-1))


# speedup vs baseline: 1.0613x; 1.0428x over previous
import functools

import jax
import jax.numpy as jnp
from jax import lax
from jax.experimental import pallas as pl
from jax.experimental.pallas import tpu as pltpu
from jax.experimental.pallas import tpu_sc as plsc

C = 8
N = 32768
TIME = 512
BATCH = 8192

NC = 2
NS = 16
NW = NC * NS
ROWS = C * BATCH
RPW = ROWS // NW
WPC = NW // C

BLK = 128
NG = RPW // BLK
NB = NG + 3
SLOTS = 7
RAWN = RPW + TIME

_mesh = plsc.VectorSubcoreMesh(core_axis_name="c", subcore_axis_name="s")


@functools.partial(
    pl.kernel,
    out_type=jax.ShapeDtypeStruct((C, BATCH, TIME), jnp.float32),
    mesh=_mesh,
    scratch_types=[
        pltpu.VMEM((RAWN,), jnp.float32),
        pltpu.VMEM((BLK, SLOTS * BLK), jnp.float32),
        pltpu.SemaphoreType.DMA,
        pltpu.SemaphoreType.DMA,
        pltpu.SemaphoreType.DMA,
        pltpu.SemaphoreType.DMA,
        pltpu.SemaphoreType.DMA,
    ],
    compiler_params=pltpu.CompilerParams(use_tc_tiling_on_sc=True),
)
def _hankel_sc(x_hbm, out_hbm, raw_v, win_v, in_sem, s0, s1, s2, s3):
    qsem = (s0, s1, s2, s3)
    wid = lax.axis_index("s") * NC + lax.axis_index("c")
    chan = wid // WPC
    b0 = (wid % WPC) * RPW
    base = chan * N + b0

    def drain(sem):
        pltpu.make_async_copy(
            win_v.at[pl.ds(0, BLK), pl.ds(0, TIME)],
            out_hbm.at[chan, pl.ds(b0, BLK)],
            sem,
        ).wait()

    pltpu.async_copy(x_hbm.at[pl.ds(base, RAWN)], raw_v, in_sem).wait()

    for A in range(NB):
        if A >= SLOTS:
            drain(qsem[(A - SLOTS) % 4])

        slot = (A % SLOTS) * BLK

        @plsc.parallel_loop(0, BLK, unroll=4)
        def build(j):
            for m in range(BLK // 16):
                win_v[j, pl.ds(slot + 16 * m, 16)] = raw_v[
                    pl.ds(j + BLK * A + 16 * m, 16)
                ]

        if A >= 3:
            g = A - 3
            row = b0 + BLK * g
            sem = qsem[g % 4]
            s = g % SLOTS
            n1 = min(SLOTS - s, 4)
            pltpu.async_copy(
                win_v.at[pl.ds(0, BLK), pl.ds(BLK * s, BLK * n1)],
                out_hbm.at[chan, pl.ds(row, BLK), pl.ds(0, BLK * n1)],
                sem,
            )
            if n1 < 4:
                pltpu.async_copy(
                    win_v.at[pl.ds(0, BLK), pl.ds(0, BLK * (4 - n1))],
                    out_hbm.at[
                        chan, pl.ds(row, BLK), pl.ds(BLK * n1, BLK * (4 - n1))
                    ],
                    sem,
                )

    for g in range(NG - 4, NG):
        drain(qsem[g % 4])


def kernel(x):
    return _hankel_sc(x.reshape(-1))
